# baseline (speedup 1.0000x reference)
import jax
import jax.numpy as jnp
from jax.experimental import pallas as pl
from jax.experimental.pallas import tpu as pltpu


def _proj_kernel(x_ref, w_ref, x1_ref):
    x1_ref[...] = jax.lax.dot_general(
        x_ref[...], w_ref[...],
        (((1,), (1,)), ((), ())),
        preferred_element_type=jnp.float32,
    ).astype(jnp.bfloat16)


def _agg_kernel(x1_ref, nb_ref, out_ref):
    out_ref[...] = jax.lax.dot(
        nb_ref[...].astype(jnp.bfloat16), x1_ref[...],
        preferred_element_type=jnp.float32,
    )


def kernel(x, neighborhood, W):
    n, d_in = x.shape
    d_out = W.shape[0]
    bm = 400
    assert n % bm == 0

    x1 = pl.pallas_call(
        _proj_kernel,
        out_shape=jax.ShapeDtypeStruct((n, d_out), jnp.bfloat16),
    )(x, W)

    return pl.pallas_call(
        _agg_kernel,
        grid=(n // bm,),
        in_specs=[
            pl.BlockSpec((n, d_out), lambda i: (0, 0)),
            pl.BlockSpec((bm, n), lambda i: (i, 0)),
        ],
        out_specs=pl.BlockSpec((bm, d_out), lambda i: (i, 0)),
        out_shape=jax.ShapeDtypeStruct((n, d_out), jnp.float32),
        compiler_params=pltpu.CompilerParams(
            dimension_semantics=("parallel",),
        ),
    )(x1, neighborhood)


# speedup vs baseline: 1.0232x; 1.0232x over previous
import jax
import jax.numpy as jnp
from jax.experimental import pallas as pl
from jax.experimental.pallas import tpu as pltpu

_BM = 400
_NCHUNK = 5
_CH = _BM // _NCHUNK


def _fused_kernel(x_ref, w_ref, nb_ref, out_ref, buf_ref, x1_ref, sems):
    i = pl.program_id(0)
    num = pl.num_programs(0)

    def start_tile(tile, slot):
        for c in range(_NCHUNK):
            pltpu.make_async_copy(
                nb_ref.at[pl.ds(tile * _BM + c * _CH, _CH), :],
                buf_ref.at[slot, pl.ds(c * _CH, _CH), :],
                sems.at[slot, c],
            ).start()

    @pl.when(i == 0)
    def _():
        start_tile(0, 0)
        x1_ref[...] = jax.lax.dot_general(
            x_ref[...], w_ref[...],
            (((1,), (1,)), ((), ())),
            preferred_element_type=jnp.float32,
        ).astype(jnp.bfloat16)

    @pl.when(i + 1 < num)
    def _():
        start_tile(i + 1, (i + 1) % 2)

    slot = i % 2
    for c in range(_NCHUNK):
        pltpu.make_async_copy(
            nb_ref.at[pl.ds(i * _BM + c * _CH, _CH), :],
            buf_ref.at[slot, pl.ds(c * _CH, _CH), :],
            sems.at[slot, c],
        ).wait()

    out_ref[...] = jax.lax.dot(
        buf_ref[slot].astype(jnp.bfloat16), x1_ref[...],
        preferred_element_type=jnp.float32,
    )


def kernel(x, neighborhood, W):
    n, d_in = x.shape
    d_out = W.shape[0]
    assert n % _BM == 0
    grid = (n // _BM,)
    return pl.pallas_call(
        _fused_kernel,
        grid=grid,
        in_specs=[
            pl.BlockSpec((n, d_in), lambda i: (0, 0)),
            pl.BlockSpec((d_out, d_in), lambda i: (0, 0)),
            pl.BlockSpec(memory_space=pltpu.MemorySpace.HBM),
        ],
        out_specs=pl.BlockSpec((_BM, d_out), lambda i: (i, 0)),
        out_shape=jax.ShapeDtypeStruct((n, d_out), jnp.float32),
        scratch_shapes=[
            pltpu.VMEM((2, _BM, n), jnp.float32),
            pltpu.VMEM((n, d_out), jnp.bfloat16),
            pltpu.SemaphoreType.DMA((2, _NCHUNK)),
        ],
        compiler_params=pltpu.CompilerParams(
            dimension_semantics=("arbitrary",),
        ),
    )(x, W, neighborhood)


# speedup vs baseline: 1.0327x; 1.0093x over previous
import jax
import jax.numpy as jnp
from jax.experimental import pallas as pl
from jax.experimental.pallas import tpu as pltpu


def _fused_kernel(x_ref, w_ref, nb_ref, out_ref, x1_ref):
    @pl.when(pl.program_id(0) == 0)
    def _():
        x1_ref[...] = jax.lax.dot_general(
            x_ref[...], w_ref[...],
            (((1,), (1,)), ((), ())),
            preferred_element_type=jnp.float32,
        )

    out_ref[...] = jax.lax.dot(
        nb_ref[...], x1_ref[...],
        preferred_element_type=jnp.float32,
    )


def kernel(x, neighborhood, W):
    n, d_in = x.shape
    d_out = W.shape[0]
    bm = 400
    assert n % bm == 0
    grid = (n // bm,)
    return pl.pallas_call(
        _fused_kernel,
        grid=grid,
        in_specs=[
            pl.BlockSpec((n, d_in), lambda i: (0, 0)),
            pl.BlockSpec((d_out, d_in), lambda i: (0, 0)),
            pl.BlockSpec((bm, n), lambda i: (i, 0)),
        ],
        out_specs=pl.BlockSpec((bm, d_out), lambda i: (i, 0)),
        out_shape=jax.ShapeDtypeStruct((n, d_out), jnp.float32),
        scratch_shapes=[pltpu.VMEM((n, d_out), jnp.float32)],
        compiler_params=pltpu.CompilerParams(
            dimension_semantics=("arbitrary",),
        ),
    )(x, W, neighborhood)


# speedup vs baseline: 1.0337x; 1.0010x over previous
import jax
import jax.numpy as jnp
from jax.experimental import pallas as pl
from jax.experimental.pallas import tpu as pltpu


def _fused_kernel(x_ref, w_ref, nb_ref, out_ref, x1_ref):
    @pl.when(pl.program_id(0) == 0)
    def _():
        x1_ref[...] = jax.lax.dot_general(
            x_ref[...], w_ref[...],
            (((1,), (1,)), ((), ())),
            preferred_element_type=jnp.float32,
        ).astype(jnp.bfloat16)

    out_ref[...] = jax.lax.dot(
        nb_ref[...], x1_ref[...],
        preferred_element_type=jnp.float32,
    )


def kernel(x, neighborhood, W):
    n, d_in = x.shape
    d_out = W.shape[0]
    bm = 400
    assert n % bm == 0
    grid = (n // bm,)
    return pl.pallas_call(
        _fused_kernel,
        grid=grid,
        in_specs=[
            pl.BlockSpec((n, d_in), lambda i: (0, 0)),
            pl.BlockSpec((d_out, d_in), lambda i: (0, 0)),
            pl.BlockSpec((bm, n), lambda i: (i, 0)),
        ],
        out_specs=pl.BlockSpec((bm, d_out), lambda i: (i, 0)),
        out_shape=jax.ShapeDtypeStruct((n, d_out), jnp.float32),
        scratch_shapes=[pltpu.VMEM((n, d_out), jnp.bfloat16)],
        compiler_params=pltpu.CompilerParams(
            dimension_semantics=("arbitrary",),
        ),
    )(x, W, neighborhood)
